# baseline (speedup 1.0000x reference)
import jax
import jax.numpy as jnp
from jax.experimental import pallas as pl

_TOPK = 8


def _stage1(ne_ref, ee_ref, inpx_ref, wf_ref, adet_ref):
    nb = ne_ref[...]
    ee = ee_ref[...]
    logits = jax.lax.dot_general(nb, ee, (((1,), (1,)), ((), ())),
                                 preferred_element_type=jnp.float32)
    logits = jnp.maximum(logits, 0.0)
    R, E = logits.shape
    iota = jax.lax.broadcasted_iota(jnp.int32, (R, E), 1)
    mx0 = jnp.max(logits, axis=-1, keepdims=True)
    work = logits
    sel = jnp.zeros((R, E), jnp.bool_)
    for k in range(_TOPK):
        mx = mx0 if k == 0 else jnp.max(work, axis=-1, keepdims=True)
        ismx = work == mx
        first = jnp.min(jnp.where(ismx, iota, E), axis=-1, keepdims=True)
        pick = iota == first
        sel = jnp.logical_or(sel, pick)
        work = jnp.where(pick, -jnp.inf, work)
    ex = jnp.where(sel, jnp.exp(logits - mx0), 0.0)
    wf = ex / jnp.sum(ex, axis=-1, keepdims=True)
    wf_ref[...] = wf
    inp8 = inpx_ref[...]
    at = jax.lax.dot_general(wf, inp8, (((0,), (1,)), ((), ())),
                             preferred_element_type=jnp.float32,
                             precision=jax.lax.Precision.HIGHEST)
    de = jax.lax.dot_general(wf, jnp.ones((1, wf.shape[0]), jnp.float32),
                             (((0,), (1,)), ((), ())),
                             preferred_element_type=jnp.float32,
                             precision=jax.lax.Precision.HIGHEST)
    adet_ref[...] = jnp.zeros_like(adet_ref)
    adet_ref[:, 0:8] = at
    adet_ref[:, 8:9] = de


def _stage2(wf_ref, adet_ref, lw_ref, lb_ref, hw_ref, hb_ref, out_ref):
    wf = wf_ref[...]
    adet = adet_ref[...]
    de = adet[:, 8:9]
    m = adet / jnp.maximum(de, 1e-6)
    ab = jax.lax.dot_general(wf, m, (((1,), (0,)), ((), ())),
                             preferred_element_type=jnp.float32,
                             precision=jax.lax.Precision.HIGHEST)
    dv = jnp.maximum(jnp.sum(wf, axis=-1, keepdims=True), 1e-6)
    u = jax.lax.dot_general(lw_ref[...], hw_ref[...], (((1,), (0,)), ((), ())),
                            preferred_element_type=jnp.float32,
                            precision=jax.lax.Precision.HIGHEST)
    v = jax.lax.dot_general(lb_ref[...], hw_ref[...], (((1,), (0,)), ((), ())),
                            preferred_element_type=jnp.float32,
                            precision=jax.lax.Precision.HIGHEST)
    hb = hb_ref[...]
    base = (ab[:, 8:9] / dv) * v + hb
    for bs in range(8):
        x = (ab[:, bs:bs + 1] / dv) * u + base
        out_ref[bs, :, :] = jnp.where(x > 0, x, jnp.exp(x) - 1.0)


def kernel(input, locations, lin_w, lin_b, node_emb, edge_emb, hgcn_w, hgcn_b):
    del locations
    B, S, N, _ = input.shape
    E, D = edge_emb.shape
    BS = B * S
    R = 1000
    nblk = N // R
    inp8 = input.reshape(BS, N)

    wf, adet = pl.pallas_call(
        _stage1,
        out_shape=[
            jax.ShapeDtypeStruct((N, E), jnp.float32),
            jax.ShapeDtypeStruct((E, 16), jnp.float32),
        ],
    )(node_emb, edge_emb, inp8)

    out = pl.pallas_call(
        _stage2,
        grid=(nblk,),
        in_specs=[
            pl.BlockSpec((R, E), lambda j: (j, 0)),
            pl.BlockSpec((E, 16), lambda j: (0, 0)),
            pl.BlockSpec((1, D), lambda j: (0, 0)),
            pl.BlockSpec((1, D), lambda j: (0, 0)),
            pl.BlockSpec((D, D), lambda j: (0, 0)),
            pl.BlockSpec((1, D), lambda j: (0, 0)),
        ],
        out_specs=pl.BlockSpec((BS, R, D), lambda j: (0, j, 0)),
        out_shape=jax.ShapeDtypeStruct((BS, N, D), jnp.float32),
    )(wf, adet, lin_w, lin_b.reshape(1, D), hgcn_w, hgcn_b.reshape(1, D))

    return out.reshape(B, S, N, D)


# speedup vs baseline: 1.0891x; 1.0891x over previous
import jax
import jax.numpy as jnp
from jax.experimental import pallas as pl
from jax.experimental.pallas import tpu as pltpu

_TOPK = 8


def _fused(ne_ref, ee_ref, inp8_ref, lw_ref, lb_ref, hw_ref, hb_ref,
           out_ref, wf_s, adet_s):
    p = pl.program_id(0)
    j = pl.program_id(1)
    R = ne_ref.shape[0]

    @pl.when(p == 0)
    def _phase0():
        nb = ne_ref[...]
        ee = ee_ref[...]
        logits = jax.lax.dot_general(nb, ee, (((1,), (1,)), ((), ())),
                                     preferred_element_type=jnp.float32)
        logits = jnp.maximum(logits, 0.0)
        Rr, E = logits.shape
        iota = jax.lax.broadcasted_iota(jnp.int32, (Rr, E), 1)
        mx0 = jnp.max(logits, axis=-1, keepdims=True)
        work = logits
        sel = jnp.zeros((Rr, E), jnp.bool_)
        for k in range(_TOPK):
            mx = mx0 if k == 0 else jnp.max(work, axis=-1, keepdims=True)
            ismx = work == mx
            first = jnp.min(jnp.where(ismx, iota, E), axis=-1, keepdims=True)
            pick = iota == first
            sel = jnp.logical_or(sel, pick)
            work = jnp.where(pick, -jnp.inf, work)
        ex = jnp.where(sel, jnp.exp(logits - mx0), 0.0)
        wf = ex / jnp.sum(ex, axis=-1, keepdims=True)
        wf_s[pl.ds(j * R, R), :] = wf

    @pl.when(jnp.logical_and(p == 1, j == 0))
    def _edge_scalars():
        wfull = wf_s[...]
        inp8 = inp8_ref[...]
        at = jax.lax.dot_general(wfull, inp8, (((0,), (1,)), ((), ())),
                                 preferred_element_type=jnp.float32,
                                 precision=jax.lax.Precision.HIGHEST)
        de = jax.lax.dot_general(wfull, jnp.ones((1, wfull.shape[0]), jnp.float32),
                                 (((0,), (1,)), ((), ())),
                                 preferred_element_type=jnp.float32,
                                 precision=jax.lax.Precision.HIGHEST)
        adet_s[...] = jnp.zeros_like(adet_s)
        adet_s[:, 0:8] = at
        adet_s[:, 8:9] = de

    @pl.when(p == 1)
    def _phase1():
        wf = wf_s[pl.ds(j * R, R), :]
        adet = adet_s[...]
        de = adet[:, 8:9]
        m = adet / jnp.maximum(de, 1e-6)
        ab = jax.lax.dot_general(wf, m, (((1,), (0,)), ((), ())),
                                 preferred_element_type=jnp.float32,
                                 precision=jax.lax.Precision.HIGHEST)
        dv = jnp.maximum(jnp.sum(wf, axis=-1, keepdims=True), 1e-6)
        u = jax.lax.dot_general(lw_ref[...], hw_ref[...], (((1,), (0,)), ((), ())),
                                preferred_element_type=jnp.float32,
                                precision=jax.lax.Precision.HIGHEST)
        v = jax.lax.dot_general(lb_ref[...], hw_ref[...], (((1,), (0,)), ((), ())),
                                preferred_element_type=jnp.float32,
                                precision=jax.lax.Precision.HIGHEST)
        hb = hb_ref[...]
        base = (ab[:, 8:9] / dv) * v + hb
        for bs in range(8):
            x = (ab[:, bs:bs + 1] / dv) * u + base
            out_ref[bs, :, :] = jnp.where(x > 0, x, jnp.exp(x) - 1.0)


def kernel(input, locations, lin_w, lin_b, node_emb, edge_emb, hgcn_w, hgcn_b):
    del locations
    B, S, N, _ = input.shape
    E, D = edge_emb.shape
    BS = B * S
    R = 2000
    nblk = N // R
    inp8 = input.reshape(BS, N)

    out = pl.pallas_call(
        _fused,
        grid=(2, nblk),
        in_specs=[
            pl.BlockSpec((R, D), lambda p, j: ((1 - p) * j, 0)),
            pl.BlockSpec((E, D), lambda p, j: (0, 0)),
            pl.BlockSpec((BS, N), lambda p, j: (0, 0)),
            pl.BlockSpec((1, D), lambda p, j: (0, 0)),
            pl.BlockSpec((1, D), lambda p, j: (0, 0)),
            pl.BlockSpec((D, D), lambda p, j: (0, 0)),
            pl.BlockSpec((1, D), lambda p, j: (0, 0)),
        ],
        out_specs=pl.BlockSpec((BS, R, D), lambda p, j: (0, p * j, 0)),
        out_shape=jax.ShapeDtypeStruct((BS, N, D), jnp.float32),
        scratch_shapes=[
            pltpu.VMEM((N, E), jnp.float32),
            pltpu.VMEM((E, 16), jnp.float32),
        ],
    )(node_emb, edge_emb, inp8, lin_w, lin_b.reshape(1, D), hgcn_w,
      hgcn_b.reshape(1, D))

    return out.reshape(B, S, N, D)


# speedup vs baseline: 1.5942x; 1.4638x over previous
import jax
import jax.numpy as jnp
from jax.experimental import pallas as pl
from jax.experimental.pallas import tpu as pltpu

_TOPK = 8


def _hgcn(ne_ref, ee_ref, inp8_ref, lw_ref, lb_ref, hw_ref,
          abt_ref, uv_ref, wf_s):
    j = pl.program_id(0)
    nblk = pl.num_programs(0) - 1
    R = ne_ref.shape[0]

    @pl.when(j < nblk)
    def _phase0():
        nb = ne_ref[...]
        ee = ee_ref[...]
        logits = jax.lax.dot_general(nb, ee, (((1,), (1,)), ((), ())),
                                     preferred_element_type=jnp.float32)
        logits = jnp.maximum(logits, 0.0)
        Rr, E = logits.shape
        iota = jax.lax.broadcasted_iota(jnp.int32, (Rr, E), 1)
        mx0 = jnp.max(logits, axis=-1, keepdims=True)
        work = logits
        sel = jnp.zeros((Rr, E), jnp.bool_)
        for k in range(_TOPK):
            mx = mx0 if k == 0 else jnp.max(work, axis=-1, keepdims=True)
            ismx = work == mx
            first = jnp.min(jnp.where(ismx, iota, E), axis=-1, keepdims=True)
            pick = iota == first
            sel = jnp.logical_or(sel, pick)
            work = jnp.where(pick, -jnp.inf, work)
        ex = jnp.where(sel, jnp.exp(logits - mx0), 0.0)
        wf = ex / jnp.sum(ex, axis=-1, keepdims=True)
        wf_s[pl.ds(j * R, R), :] = wf

    @pl.when(j == nblk)
    def _finalize():
        wfull = wf_s[...]
        N = wfull.shape[0]
        E = wfull.shape[1]
        inp8 = inp8_ref[...]
        at = jax.lax.dot_general(wfull, inp8, (((0,), (1,)), ((), ())),
                                 preferred_element_type=jnp.float32,
                                 precision=jax.lax.Precision.HIGHEST)
        de = jax.lax.dot_general(wfull, jnp.ones((1, N), jnp.float32),
                                 (((0,), (1,)), ((), ())),
                                 preferred_element_type=jnp.float32,
                                 precision=jax.lax.Precision.HIGHEST)
        dec = jnp.maximum(de, 1e-6)
        m = jnp.concatenate(
            [at / dec, de / dec, jnp.ones((E, 1), jnp.float32),
             jnp.zeros((E, 6), jnp.float32)], axis=1)
        abt_ref[...] = jax.lax.dot_general(m, wfull, (((0,), (1,)), ((), ())),
                                           preferred_element_type=jnp.float32,
                                           precision=jax.lax.Precision.HIGHEST)
        u = jax.lax.dot_general(lw_ref[...], hw_ref[...], (((1,), (0,)), ((), ())),
                                preferred_element_type=jnp.float32,
                                precision=jax.lax.Precision.HIGHEST)
        v = jax.lax.dot_general(lb_ref[...], hw_ref[...], (((1,), (0,)), ((), ())),
                                preferred_element_type=jnp.float32,
                                precision=jax.lax.Precision.HIGHEST)
        uv_ref[...] = jnp.concatenate(
            [u, v, jnp.zeros((6, u.shape[1]), jnp.float32)], axis=0)


def kernel(input, locations, lin_w, lin_b, node_emb, edge_emb, hgcn_w, hgcn_b):
    del locations
    B, S, N, _ = input.shape
    E, D = edge_emb.shape
    BS = B * S
    R = 2000
    nblk = N // R
    inp8 = input.reshape(BS, N)

    abt, uv = pl.pallas_call(
        _hgcn,
        grid=(nblk + 1,),
        in_specs=[
            pl.BlockSpec((R, D), lambda j, _n=nblk: ((1 - j // _n) * j, 0)),
            pl.BlockSpec((E, D), lambda j: (0, 0)),
            pl.BlockSpec((BS, N), lambda j: (0, 0)),
            pl.BlockSpec((1, D), lambda j: (0, 0)),
            pl.BlockSpec((1, D), lambda j: (0, 0)),
            pl.BlockSpec((D, D), lambda j: (0, 0)),
        ],
        out_specs=[
            pl.BlockSpec((16, N), lambda j: (0, 0)),
            pl.BlockSpec((8, D), lambda j: (0, 0)),
        ],
        out_shape=[
            jax.ShapeDtypeStruct((16, N), jnp.float32),
            jax.ShapeDtypeStruct((8, D), jnp.float32),
        ],
        scratch_shapes=[pltpu.VMEM((N, E), jnp.float32)],
    )(node_emb, edge_emb, inp8, lin_w, lin_b.reshape(1, D), hgcn_w)

    dv = jnp.maximum(abt[9], 1e-6)
    alpha = abt[0:8] / dv
    beta = abt[8] / dv
    x = (alpha[:, :, None] * uv[0]
         + (beta[:, None] * uv[1] + hgcn_b)[None, :, :])
    out = jnp.where(x > 0, x, jnp.expm1(x))
    return out.reshape(B, S, N, D)


# speedup vs baseline: 1.9963x; 1.2522x over previous
import jax
import jax.numpy as jnp
from jax.experimental import pallas as pl
from jax.experimental.pallas import tpu as pltpu

_TOPK = 8


def _hgcn(ne_ref, ee_ref, inp8_ref, lw_ref, lb_ref, hw_ref,
          abt_ref, uv_ref, wf_s):
    j = pl.program_id(0)
    nblk = pl.num_programs(0) - 1
    R = ne_ref.shape[0]

    @pl.when(j < nblk)
    def _phase0():
        nb = ne_ref[...]
        ee = ee_ref[...]
        logits = jax.lax.dot_general(nb, ee, (((1,), (1,)), ((), ())),
                                     preferred_element_type=jnp.float32)
        logits = jnp.maximum(logits, 0.0)
        Rr, E = logits.shape
        mx0 = jnp.max(logits, axis=-1, keepdims=True)
        work = logits
        sel = jnp.zeros((Rr, E), jnp.bool_)
        neg = jnp.float32(-jnp.inf)
        for k in range(_TOPK):
            mx = mx0 if k == 0 else jnp.max(work, axis=-1, keepdims=True)
            ismx = jnp.logical_and(work == mx, mx != neg)
            sel = jnp.logical_or(sel, ismx)
            work = jnp.where(ismx, neg, work)
        nsel = jnp.sum(jnp.where(sel, 1.0, 0.0), axis=-1, keepdims=True)
        dirty = jnp.sum(jnp.where(nsel != float(_TOPK), 1.0, 0.0))

        def _write(sel_f):
            ex = jnp.where(sel_f, jnp.exp(logits - mx0), 0.0)
            wf = ex / jnp.sum(ex, axis=-1, keepdims=True)
            wf_s[pl.ds(j * R, R), :] = wf

        @pl.when(dirty == 0.0)
        def _fast():
            _write(sel)

        @pl.when(dirty != 0.0)
        def _exact():
            iota = jax.lax.broadcasted_iota(jnp.int32, (Rr, E), 1)
            work2 = logits
            sel2 = jnp.zeros((Rr, E), jnp.bool_)
            for k in range(_TOPK):
                mx = jnp.max(work2, axis=-1, keepdims=True)
                ismx = work2 == mx
                first = jnp.min(jnp.where(ismx, iota, E), axis=-1,
                                keepdims=True)
                pick = iota == first
                sel2 = jnp.logical_or(sel2, pick)
                work2 = jnp.where(pick, neg, work2)
            _write(sel2)

    @pl.when(j == nblk)
    def _finalize():
        wfull = wf_s[...]
        N = wfull.shape[0]
        E = wfull.shape[1]
        inp8 = inp8_ref[...]
        at = jax.lax.dot_general(wfull, inp8, (((0,), (1,)), ((), ())),
                                 preferred_element_type=jnp.float32,
                                 precision=jax.lax.Precision.HIGHEST)
        de = jax.lax.dot_general(wfull, jnp.ones((1, N), jnp.float32),
                                 (((0,), (1,)), ((), ())),
                                 preferred_element_type=jnp.float32,
                                 precision=jax.lax.Precision.HIGHEST)
        dec = jnp.maximum(de, 1e-6)
        m = jnp.concatenate(
            [at / dec, de / dec, jnp.ones((E, 1), jnp.float32),
             jnp.zeros((E, 6), jnp.float32)], axis=1)
        abt_ref[...] = jax.lax.dot_general(m, wfull, (((0,), (1,)), ((), ())),
                                           preferred_element_type=jnp.float32,
                                           precision=jax.lax.Precision.HIGHEST)
        u = jax.lax.dot_general(lw_ref[...], hw_ref[...], (((1,), (0,)), ((), ())),
                                preferred_element_type=jnp.float32,
                                precision=jax.lax.Precision.HIGHEST)
        v = jax.lax.dot_general(lb_ref[...], hw_ref[...], (((1,), (0,)), ((), ())),
                                preferred_element_type=jnp.float32,
                                precision=jax.lax.Precision.HIGHEST)
        uv_ref[...] = jnp.concatenate(
            [u, v, jnp.zeros((6, u.shape[1]), jnp.float32)], axis=0)


def kernel(input, locations, lin_w, lin_b, node_emb, edge_emb, hgcn_w, hgcn_b):
    del locations
    B, S, N, _ = input.shape
    E, D = edge_emb.shape
    BS = B * S
    R = 2000
    nblk = N // R
    inp8 = input.reshape(BS, N)

    abt, uv = pl.pallas_call(
        _hgcn,
        grid=(nblk + 1,),
        in_specs=[
            pl.BlockSpec((R, D), lambda j, _n=nblk: ((1 - j // _n) * j, 0)),
            pl.BlockSpec((E, D), lambda j: (0, 0)),
            pl.BlockSpec((BS, N), lambda j: (0, 0)),
            pl.BlockSpec((1, D), lambda j: (0, 0)),
            pl.BlockSpec((1, D), lambda j: (0, 0)),
            pl.BlockSpec((D, D), lambda j: (0, 0)),
        ],
        out_specs=[
            pl.BlockSpec((16, N), lambda j: (0, 0)),
            pl.BlockSpec((8, D), lambda j: (0, 0)),
        ],
        out_shape=[
            jax.ShapeDtypeStruct((16, N), jnp.float32),
            jax.ShapeDtypeStruct((8, D), jnp.float32),
        ],
        scratch_shapes=[pltpu.VMEM((N, E), jnp.float32)],
    )(node_emb, edge_emb, inp8, lin_w, lin_b.reshape(1, D), hgcn_w)

    dv = jnp.maximum(abt[9], 1e-6)
    alpha = abt[0:8] / dv
    beta = abt[8] / dv
    x = (alpha[:, :, None] * uv[0]
         + (beta[:, None] * uv[1] + hgcn_b)[None, :, :])
    out = jnp.where(x > 0, x, jnp.expm1(x))
    return out.reshape(B, S, N, D)
